# baseline (speedup 1.0000x reference)
import functools

import jax
import jax.numpy as jnp
from jax.experimental import pallas as pl
from jax.experimental.pallas import tpu as pltpu

_BIG = 1e30


def _chamfer_body(a_ref, ga_ref, m_ref, c_ref, out_p_ref, out_g_ref,
                  colmin_ref, *, num_i):
    i = pl.program_id(1)
    b = pl.program_id(0)

    A = a_ref[0]
    GA = ga_ref[0]
    m = m_ref[0]
    conf = c_ref[0]

    D = jax.lax.dot_general(A, GA, (((1,), (0,)), ((), ())),
                            preferred_element_type=jnp.float32)

    rowmin = jnp.min(D, axis=1, keepdims=True)
    safe_conf = jnp.where(m > 0, conf, 1.0)
    step_p = jnp.sum(jnp.maximum(rowmin, 0.0) * conf * m
                     - jnp.log(safe_conf) * m)

    cmin = jnp.min(D, axis=0, keepdims=True)

    @pl.when(i == 0)
    def _():
        colmin_ref[...] = cmin

    @pl.when(i > 0)
    def _():
        colmin_ref[...] = jnp.minimum(colmin_ref[...], cmin)

    @pl.when((i == 0) & (b == 0))
    def _():
        out_p_ref[...] = jnp.zeros_like(out_p_ref)
        out_g_ref[...] = jnp.zeros_like(out_g_ref)

    out_p_ref[...] += jnp.full((1, 1), step_p, jnp.float32)

    @pl.when(i == num_i - 1)
    def _():
        out_g_ref[...] += jnp.full(
            (1, 1), jnp.sum(jnp.maximum(colmin_ref[...], 0.0)), jnp.float32)


def kernel(x_gt, x_pred, mask, confidence):
    B, V1, _ = x_gt.shape
    V2 = x_pred.shape[1]
    TI = 1024
    num_i = V2 // TI

    m = jnp.squeeze(mask, -1).astype(jnp.float32)
    pn = jnp.sum(x_pred * x_pred, axis=-1)
    gn = jnp.sum(x_gt * x_gt, axis=-1)
    pnb = pn + (1.0 - m) * _BIG

    ones_p = jnp.ones((B, V2, 1), jnp.float32)
    zeros_p = jnp.zeros((B, V2, 3), jnp.float32)
    a_aug = jnp.concatenate(
        [x_pred * (-2.0), pnb[..., None], ones_p, zeros_p], axis=-1)

    ones_g = jnp.ones((B, V1, 1), jnp.float32)
    zeros_g = jnp.zeros((B, V1, 3), jnp.float32)
    ga_aug = jnp.swapaxes(
        jnp.concatenate([x_gt, ones_g, gn[..., None], zeros_g], axis=-1),
        1, 2)

    m3 = m[..., None]
    c3 = confidence[..., None]

    out_p, out_g = pl.pallas_call(
        functools.partial(_chamfer_body, num_i=num_i),
        grid=(B, num_i),
        in_specs=[
            pl.BlockSpec((1, TI, 8), lambda b, i: (b, i, 0)),
            pl.BlockSpec((1, 8, V1), lambda b, i: (b, 0, 0)),
            pl.BlockSpec((1, TI, 1), lambda b, i: (b, i, 0)),
            pl.BlockSpec((1, TI, 1), lambda b, i: (b, i, 0)),
        ],
        out_specs=[
            pl.BlockSpec((1, 1), lambda b, i: (0, 0)),
            pl.BlockSpec((1, 1), lambda b, i: (0, 0)),
        ],
        out_shape=[
            jax.ShapeDtypeStruct((1, 1), jnp.float32),
            jax.ShapeDtypeStruct((1, 1), jnp.float32),
        ],
        scratch_shapes=[pltpu.VMEM((1, V1), jnp.float32)],
    )(a_aug, ga_aug, m3, c3)

    return out_p[0, 0] / (B * V2) + out_g[0, 0] / (B * V1)


# speedup vs baseline: 1.1410x; 1.1410x over previous
import functools

import jax
import jax.numpy as jnp
from jax.experimental import pallas as pl
from jax.experimental.pallas import tpu as pltpu

_BIG = 1e30


def _chamfer_body(g_ref, p_ref, pbias_ref, m_ref, c_ref,
                  out_p_ref, out_g_ref, predmin_ref, *, num_j):
    j = pl.program_id(1)
    b = pl.program_id(0)

    G = g_ref[0]
    P = p_ref[0]
    pbias = pbias_ref[0]

    gn = jnp.sum(G * G, axis=1, keepdims=True)
    E = jax.lax.dot_general(G * (-2.0), P, (((1,), (1,)), ((), ())),
                            preferred_element_type=jnp.float32)
    D = E + gn + pbias

    gmin = jnp.min(D, axis=1, keepdims=True)
    step_g = jnp.sum(jnp.maximum(gmin, 0.0))

    pmin = jnp.min(D, axis=0, keepdims=True)

    @pl.when(j == 0)
    def _():
        predmin_ref[...] = pmin

    @pl.when(j > 0)
    def _():
        predmin_ref[...] = jnp.minimum(predmin_ref[...], pmin)

    @pl.when((j == 0) & (b == 0))
    def _():
        out_p_ref[...] = jnp.zeros_like(out_p_ref)
        out_g_ref[...] = jnp.zeros_like(out_g_ref)

    out_g_ref[...] += jnp.full((1, 1), step_g, jnp.float32)

    @pl.when(j == num_j - 1)
    def _():
        m = m_ref[0]
        conf = c_ref[0]
        safe_conf = jnp.where(m > 0, conf, 1.0)
        loss_p = (jnp.maximum(predmin_ref[...], 0.0) * conf * m
                  - jnp.log(safe_conf) * m)
        out_p_ref[...] += jnp.full((1, 1), jnp.sum(loss_p), jnp.float32)


def kernel(x_gt, x_pred, mask, confidence):
    B, V1, _ = x_gt.shape
    V2 = x_pred.shape[1]
    TJ = 1024
    num_j = V1 // TJ

    m = jnp.squeeze(mask, -1).astype(jnp.float32)
    pn = jnp.sum(x_pred * x_pred, axis=-1)
    pbias = pn + (1.0 - m) * _BIG

    out_p, out_g = pl.pallas_call(
        functools.partial(_chamfer_body, num_j=num_j),
        grid=(B, num_j),
        in_specs=[
            pl.BlockSpec((1, TJ, 3), lambda b, j: (b, j, 0)),
            pl.BlockSpec((1, V2, 3), lambda b, j: (b, 0, 0)),
            pl.BlockSpec((1, 1, V2), lambda b, j: (b, 0, 0)),
            pl.BlockSpec((1, 1, V2), lambda b, j: (b, 0, 0)),
            pl.BlockSpec((1, 1, V2), lambda b, j: (b, 0, 0)),
        ],
        out_specs=[
            pl.BlockSpec((1, 1), lambda b, j: (0, 0)),
            pl.BlockSpec((1, 1), lambda b, j: (0, 0)),
        ],
        out_shape=[
            jax.ShapeDtypeStruct((1, 1), jnp.float32),
            jax.ShapeDtypeStruct((1, 1), jnp.float32),
        ],
        scratch_shapes=[pltpu.VMEM((1, V2), jnp.float32)],
    )(x_gt, x_pred, pbias[:, None, :], m[:, None, :], confidence[:, None, :])

    return out_p[0, 0] / (B * V2) + out_g[0, 0] / (B * V1)


# speedup vs baseline: 1.1991x; 1.0509x over previous
import functools

import jax
import jax.numpy as jnp
from jax.experimental import pallas as pl
from jax.experimental.pallas import tpu as pltpu

_BIG = 1e30


def _chamfer_body(g_ref, p_ref, pbias_ref, m_ref, c_ref,
                  out_p_ref, out_g_ref, predmin_ref, *, num_j):
    j = pl.program_id(1)
    b = pl.program_id(0)

    G = g_ref[0]
    P = p_ref[0]
    pbias = pbias_ref[0]

    gn = jnp.sum(G * G, axis=1, keepdims=True)
    E = jax.lax.dot_general(G * (-2.0), P, (((1,), (1,)), ((), ())),
                            preferred_element_type=jnp.float32)
    D = E + gn + pbias

    gmin = jnp.min(D, axis=1, keepdims=True)
    step_g = jnp.sum(jnp.maximum(gmin, 0.0))

    pmin = jnp.min(D, axis=0, keepdims=True)

    @pl.when(j == 0)
    def _():
        predmin_ref[...] = pmin

    @pl.when(j > 0)
    def _():
        predmin_ref[...] = jnp.minimum(predmin_ref[...], pmin)

    @pl.when((j == 0) & (b == 0))
    def _():
        out_p_ref[...] = jnp.zeros_like(out_p_ref)
        out_g_ref[...] = jnp.zeros_like(out_g_ref)

    out_g_ref[...] += jnp.full((1, 1), step_g, jnp.float32)

    @pl.when(j == num_j - 1)
    def _():
        m = m_ref[0]
        conf = c_ref[0]
        safe_conf = jnp.where(m > 0, conf, 1.0)
        loss_p = (jnp.maximum(predmin_ref[...], 0.0) * conf * m
                  - jnp.log(safe_conf) * m)
        out_p_ref[...] += jnp.full((1, 1), jnp.sum(loss_p), jnp.float32)


def kernel(x_gt, x_pred, mask, confidence):
    B, V1, _ = x_gt.shape
    V2 = x_pred.shape[1]
    TJ = 2048
    num_j = V1 // TJ

    m = jnp.squeeze(mask, -1).astype(jnp.float32)
    pn = jnp.sum(x_pred * x_pred, axis=-1)
    pbias = pn + (1.0 - m) * _BIG

    out_p, out_g = pl.pallas_call(
        functools.partial(_chamfer_body, num_j=num_j),
        grid=(B, num_j),
        in_specs=[
            pl.BlockSpec((1, TJ, 3), lambda b, j: (b, j, 0)),
            pl.BlockSpec((1, V2, 3), lambda b, j: (b, 0, 0)),
            pl.BlockSpec((1, 1, V2), lambda b, j: (b, 0, 0)),
            pl.BlockSpec((1, 1, V2), lambda b, j: (b, 0, 0)),
            pl.BlockSpec((1, 1, V2), lambda b, j: (b, 0, 0)),
        ],
        out_specs=[
            pl.BlockSpec((1, 1), lambda b, j: (0, 0)),
            pl.BlockSpec((1, 1), lambda b, j: (0, 0)),
        ],
        out_shape=[
            jax.ShapeDtypeStruct((1, 1), jnp.float32),
            jax.ShapeDtypeStruct((1, 1), jnp.float32),
        ],
        scratch_shapes=[pltpu.VMEM((1, V2), jnp.float32)],
    )(x_gt, x_pred, pbias[:, None, :], m[:, None, :], confidence[:, None, :])

    return out_p[0, 0] / (B * V2) + out_g[0, 0] / (B * V1)
